# Initial kernel scaffold; baseline (speedup 1.0000x reference)
import functools

import jax
import jax.numpy as jnp
from jax import lax
from jax.experimental import pallas as pl
from jax.experimental.pallas import tpu as pltpu
from jax.experimental.pallas import tpu_sc as plsc

N_LAT = 32
N_OUT = 32
LANES = 16


def _build(B, N_SAMPLE):
    info = plsc.get_sparse_core_info()
    NC, NS = info.num_cores, info.num_subcores
    NW = NC * NS
    assert B % NW == 0
    RPW = B // NW
    C = 8
    NCHUNK = RPW // C

    mesh = plsc.VectorSubcoreMesh(core_axis_name="c", subcore_axis_name="s")

    @functools.partial(
        pl.kernel,
        mesh=mesh,
        out_type=jax.ShapeDtypeStruct((B, N_OUT), jnp.float32),
        scratch_types=[
            pltpu.VMEM((RPW,), jnp.int32),
            pltpu.VMEM((C, N_LAT * N_OUT), jnp.float32),
            pltpu.VMEM((C, N_OUT), jnp.float32),
            pltpu.VMEM((C, N_LAT), jnp.float32),
            pltpu.VMEM((C, N_OUT), jnp.float32),
            pltpu.SemaphoreType.DMA,
            pltpu.SemaphoreType.DMA,
        ],
    )
    def k(u_hbm, si_hbm, amat_hbm, offs_hbm, out_hbm,
          idx_v, as_v, og_v, u_v, out_v, sem_a, sem_o):
        wid = lax.axis_index("s") * NC + lax.axis_index("c")
        base = wid * RPW
        pltpu.sync_copy(si_hbm.at[pl.ds(base, RPW)], idx_v)

        def chunk(g, carry):
            off = g * C
            pltpu.async_copy(amat_hbm.at[idx_v.at[pl.ds(off, C)]], as_v, sem_a).wait()
            pltpu.async_copy(offs_hbm.at[idx_v.at[pl.ds(off, C)]], og_v, sem_o).wait()
            pltpu.sync_copy(u_hbm.at[pl.ds(base + off, C)], u_v)
            for r in range(C):
                acc0 = u_v[r, pl.ds(0, LANES)] + og_v[r, pl.ds(0, LANES)]
                acc1 = u_v[r, pl.ds(LANES, LANES)] + og_v[r, pl.ds(LANES, LANES)]
                for l in range(N_LAT):
                    ul = u_v[r, l]
                    acc0 = acc0 + ul * as_v[r, pl.ds(l * N_OUT, LANES)]
                    acc1 = acc1 + ul * as_v[r, pl.ds(l * N_OUT + LANES, LANES)]
                out_v[r, pl.ds(0, LANES)] = acc0
                out_v[r, pl.ds(LANES, LANES)] = acc1
            pltpu.sync_copy(out_v, out_hbm.at[pl.ds(base + off, C)])
            return carry

        lax.fori_loop(0, NCHUNK, chunk, 0)

    return k


def kernel(u, sample_index, amat_sample, offsets):
    B = u.shape[0]
    n_sample = amat_sample.shape[0]
    si = sample_index.squeeze() if sample_index.ndim > 1 else sample_index
    amat2 = amat_sample.reshape(n_sample, N_LAT * N_OUT)
    k = _build(B, n_sample)
    return k(u, si.astype(jnp.int32), amat2, offsets)


# speedup vs baseline: 6.1442x; 6.1442x over previous
import functools

import jax
import jax.numpy as jnp
from jax import lax
from jax.experimental import pallas as pl
from jax.experimental.pallas import tpu as pltpu
from jax.experimental.pallas import tpu_sc as plsc

N_LAT = 32
N_OUT = 32
LANES = 16


def _build(B, N_SAMPLE):
    info = plsc.get_sparse_core_info()
    NC, NS = info.num_cores, info.num_subcores
    NW = NC * NS
    assert B % NW == 0
    RPW = B // NW
    C = 8
    NCHUNK = RPW // C

    mesh = plsc.VectorSubcoreMesh(core_axis_name="c", subcore_axis_name="s")

    @functools.partial(
        pl.kernel,
        mesh=mesh,
        out_type=jax.ShapeDtypeStruct((B, N_OUT), jnp.float32),
        scratch_types=[
            pltpu.VMEM((RPW,), jnp.int32),
            pltpu.VMEM((C, N_LAT * N_OUT), jnp.float32),
            pltpu.VMEM((C, 128), jnp.float32),
            pltpu.VMEM((C, N_LAT), jnp.float32),
            pltpu.VMEM((C, N_OUT), jnp.float32),
            pltpu.SemaphoreType.DMA,
            pltpu.SemaphoreType.DMA,
        ],
    )
    def k(u_hbm, si_hbm, amat_hbm, offs_hbm, out_hbm,
          idx_v, as_v, og_v, u_v, out_v, sem_a, sem_o):
        wid = lax.axis_index("s") * NC + lax.axis_index("c")
        base = wid * RPW
        pltpu.sync_copy(si_hbm.at[pl.ds(base, RPW)], idx_v)

        def chunk(g, carry):
            off = g * C
            pltpu.async_copy(amat_hbm.at[idx_v.at[pl.ds(off, C)]], as_v, sem_a).wait()
            pltpu.async_copy(offs_hbm.at[idx_v.at[pl.ds(off, C)]], og_v, sem_o).wait()
            pltpu.sync_copy(u_hbm.at[pl.ds(base + off, C)], u_v)
            for r in range(C):
                uv0 = u_v[r, pl.ds(0, LANES)]
                uv1 = u_v[r, pl.ds(LANES, LANES)]
                acc0 = uv0 + og_v[r, pl.ds(0, LANES)]
                acc1 = uv1 + og_v[r, pl.ds(LANES, LANES)]
                for l in range(N_LAT):
                    ul = (uv0 if l < LANES else uv1)[l % LANES]
                    acc0 = acc0 + ul * as_v[r, pl.ds(l * N_OUT, LANES)]
                    acc1 = acc1 + ul * as_v[r, pl.ds(l * N_OUT + LANES, LANES)]
                out_v[r, pl.ds(0, LANES)] = acc0
                out_v[r, pl.ds(LANES, LANES)] = acc1
            pltpu.sync_copy(out_v, out_hbm.at[pl.ds(base + off, C)])
            return carry

        lax.fori_loop(0, NCHUNK, chunk, 0)

    return k


def kernel(u, sample_index, amat_sample, offsets):
    B = u.shape[0]
    n_sample = amat_sample.shape[0]
    si = sample_index.squeeze() if sample_index.ndim > 1 else sample_index
    amat2 = amat_sample.reshape(n_sample, N_LAT * N_OUT)
    offs_pad = jnp.pad(offsets, ((0, 0), (0, 128 - N_OUT)))
    k = _build(B, n_sample)
    return k(u, si.astype(jnp.int32), amat2, offs_pad)


# speedup vs baseline: 6.1791x; 1.0057x over previous
import functools

import jax
import jax.numpy as jnp
from jax import lax
from jax.experimental import pallas as pl
from jax.experimental.pallas import tpu as pltpu
from jax.experimental.pallas import tpu_sc as plsc

N_LAT = 32
N_OUT = 32
LANES = 16


def _build(B, N_SAMPLE):
    info = plsc.get_sparse_core_info()
    NC, NS = info.num_cores, info.num_subcores
    NW = NC * NS
    assert B % NW == 0
    RPW = B // NW
    C = 8
    NCHUNK = RPW // C

    mesh = plsc.VectorSubcoreMesh(core_axis_name="c", subcore_axis_name="s")

    @functools.partial(
        pl.kernel,
        mesh=mesh,
        out_type=jax.ShapeDtypeStruct((B, N_OUT), jnp.float32),
        scratch_types=[
            pltpu.VMEM((RPW,), jnp.int32),
            pltpu.VMEM((C, 8, 128), jnp.float32),
            pltpu.VMEM((C, 128), jnp.float32),
            pltpu.VMEM((C, N_LAT), jnp.float32),
            pltpu.VMEM((C, N_OUT), jnp.float32),
            pltpu.SemaphoreType.DMA,
            pltpu.SemaphoreType.DMA,
        ],
    )
    def k(u_hbm, si_hbm, amat_hbm, offs_hbm, out_hbm,
          idx_v, as_v, og_v, u_v, out_v, sem_a, sem_o):
        wid = lax.axis_index("s") * NC + lax.axis_index("c")
        base = wid * RPW
        pltpu.sync_copy(si_hbm.at[pl.ds(base, RPW)], idx_v)

        def chunk(g, carry):
            off = g * C
            pltpu.async_copy(amat_hbm.at[idx_v.at[pl.ds(off, C)]], as_v, sem_a).wait()
            pltpu.async_copy(offs_hbm.at[idx_v.at[pl.ds(off, C)]], og_v, sem_o).wait()
            pltpu.sync_copy(u_hbm.at[pl.ds(base + off, C)], u_v)
            for r in range(C):
                uv0 = u_v[r, pl.ds(0, LANES)]
                uv1 = u_v[r, pl.ds(LANES, LANES)]
                acc0 = uv0 + og_v[r, pl.ds(0, LANES)]
                acc1 = uv1 + og_v[r, pl.ds(LANES, LANES)]
                for l in range(N_LAT):
                    ul = (uv0 if l < LANES else uv1)[l % LANES]
                    acc0 = acc0 + ul * as_v[r, l // 4, pl.ds((l % 4) * N_OUT, LANES)]
                    acc1 = acc1 + ul * as_v[r, l // 4, pl.ds((l % 4) * N_OUT + LANES, LANES)]
                out_v[r, pl.ds(0, LANES)] = acc0
                out_v[r, pl.ds(LANES, LANES)] = acc1
            pltpu.sync_copy(out_v, out_hbm.at[pl.ds(base + off, C)])
            return carry

        lax.fori_loop(0, NCHUNK, chunk, 0)

    return k


def kernel(u, sample_index, amat_sample, offsets):
    B = u.shape[0]
    n_sample = amat_sample.shape[0]
    si = sample_index.squeeze() if sample_index.ndim > 1 else sample_index
    offs_pad = jnp.pad(offsets, ((0, 0), (0, 128 - N_OUT)))
    amat3 = amat_sample.reshape(n_sample, 8, 128)
    k = _build(B, n_sample)
    return k(u, si.astype(jnp.int32), amat3, offs_pad)
